# Initial kernel scaffold; baseline (speedup 1.0000x reference)
import jax
import jax.numpy as jnp
from jax.experimental import pallas as pl


def kernel(x_paper, x_author, edge_index_cites, edge_index_writes, edge_index_written_by, Wl_cites, bl_cites, Wr_cites, br_cites, Wl_writes, bl_writes, Wr_writes, br_writes, Wl_wb, bl_wb, Wr_wb, br_wb):
    raise NotImplementedError("write your pallas kernel here")


# speedup vs baseline: 1.0095x; 1.0095x over previous
import functools

import jax
import jax.numpy as jnp
from jax import lax
from jax.experimental import pallas as pl
from jax.experimental.pallas import tpu as pltpu
from jax.experimental.pallas import tpu_sc as plsc

N_NODE = 10000
D = 128
NC = 2
NS = 16
NW = NC * NS
L = 16
CHUNK = 64
PAD_N = 10112
ROWS_PER_SUB = PAD_N // NS
CROWS = 1280
CROWS_PER_SUB = CROWS // NS

CHUNKS_CITES = 157
CHUNKS_WRITES = 79


def _pad_edges(src, dst, n_chunks):
    epad = NW * n_chunks * CHUNK
    pad = epad - src.shape[0]
    src = jnp.concatenate([src, jnp.zeros((pad,), jnp.int32)])
    dst = jnp.concatenate([dst, jnp.full((pad,), N_NODE, jnp.int32)])
    return src, dst


def _pieces(total, step):
    out, k = [], 0
    while k < total:
        out.append((k, min(step, total - k)))
        k += step
    return out


def _sc_segment_sums(src_c, dst_c, src_w, dst_w, src_b, dst_b, xp, xa,
                     zfeat, onehot8):
    mesh = plsc.VectorSubcoreMesh(core_axis_name="c", subcore_axis_name="s")
    f32 = jnp.float32

    @functools.partial(
        pl.kernel,
        mesh=mesh,
        out_type=[
            jax.ShapeDtypeStruct((NC * PAD_N, D), f32),
            jax.ShapeDtypeStruct((NC * CROWS, D), f32),
            jax.ShapeDtypeStruct((NC * PAD_N, D), f32),
            jax.ShapeDtypeStruct((NC * CROWS, D), f32),
            jax.ShapeDtypeStruct((NC * PAD_N, D), f32),
            jax.ShapeDtypeStruct((NC * CROWS, D), f32),
        ],
        scratch_types=[
            pltpu.VMEM((CHUNK,), jnp.int32),
            pltpu.VMEM((CHUNK,), jnp.int32),
            pltpu.VMEM((CHUNK,), jnp.int32),
            pltpu.VMEM((CHUNK,), jnp.int32),
            pltpu.VMEM((CHUNK, D), f32),
            pltpu.VMEM((CHUNK, D), f32),
            pltpu.VMEM_SHARED((PAD_N, D), f32),
            pltpu.VMEM_SHARED((CROWS, D), f32),
            pltpu.SemaphoreType.DMA,
        ],
    )
    def sc_kernel(src_c_h, dst_c_h, src_w_h, dst_w_h, src_b_h, dst_b_h,
                  xp_h, xa_h, zfeat_h, onehot8_h,
                  o_sum_c, o_cnt_c, o_sum_w, o_cnt_w, o_sum_b, o_cnt_b,
                  sidx, didx, drow, dlane, rows, crows_v, accum, cnt_acc,
                  sem):
        c = lax.axis_index("c")
        s = lax.axis_index("s")
        wid = c * NS + s
        r0 = s * ROWS_PER_SUB
        q0 = s * CROWS_PER_SUB
        fpieces = _pieces(ROWS_PER_SUB, CHUNK)
        cpieces = _pieces(CROWS_PER_SUB, CHUNK)

        def run_relation(src_h, dst_h, x_h, out_sum, out_cnt, n_chunks):
            pltpu.sync_copy(zfeat_h, rows)
            for (k, sz) in fpieces:
                pltpu.sync_copy(rows.at[pl.ds(0, sz)],
                                accum.at[pl.ds(r0 + k, sz)])
            for (k, sz) in cpieces:
                pltpu.sync_copy(rows.at[pl.ds(0, sz)],
                                cnt_acc.at[pl.ds(q0 + k, sz)])
            plsc.subcore_barrier()
            base = wid * (n_chunks * CHUNK)

            def body(j, carry):
                off = pl.multiple_of(base + j * CHUNK, 8)
                pltpu.sync_copy(src_h.at[pl.ds(off, CHUNK)], sidx)
                pltpu.sync_copy(dst_h.at[pl.ds(off, CHUNK)], didx)
                for t in range(CHUNK // L):
                    dv = didx[pl.ds(t * L, L)]
                    drow[pl.ds(t * L, L)] = jax.lax.shift_right_logical(dv, 3)
                    dlane[pl.ds(t * L, L)] = jax.lax.bitwise_and(dv, 7)
                pltpu.async_copy(x_h.at[sidx], rows, sem).wait()
                pltpu.sync_copy(rows, accum.at[didx], add=True)
                pltpu.async_copy(onehot8_h.at[dlane], crows_v, sem).wait()
                pltpu.sync_copy(crows_v, cnt_acc.at[drow], add=True)
                return carry

            lax.fori_loop(0, n_chunks, body, 0)
            plsc.subcore_barrier()
            for (k, sz) in fpieces:
                pltpu.sync_copy(accum.at[pl.ds(r0 + k, sz)],
                                rows.at[pl.ds(0, sz)])
                pltpu.sync_copy(rows.at[pl.ds(0, sz)],
                                out_sum.at[pl.ds(c * PAD_N + r0 + k, sz)])
            for (k, sz) in cpieces:
                pltpu.sync_copy(cnt_acc.at[pl.ds(q0 + k, sz)],
                                rows.at[pl.ds(0, sz)])
                pltpu.sync_copy(rows.at[pl.ds(0, sz)],
                                out_cnt.at[pl.ds(c * CROWS + q0 + k, sz)])
            plsc.subcore_barrier()

        run_relation(src_c_h, dst_c_h, xp_h, o_sum_c, o_cnt_c, CHUNKS_CITES)
        run_relation(src_w_h, dst_w_h, xa_h, o_sum_w, o_cnt_w, CHUNKS_WRITES)
        run_relation(src_b_h, dst_b_h, xp_h, o_sum_b, o_cnt_b, CHUNKS_WRITES)

    return sc_kernel(src_c, dst_c, src_w, dst_w, src_b, dst_b, xp, xa,
                     zfeat, onehot8)


def _tc_epilogue_body(sc0_ref, sc1_ref, sw0_ref, sw1_ref, sb0_ref, sb1_ref,
                      cc0_ref, cc1_ref, cw0_ref, cw1_ref, cb0_ref, cb1_ref,
                      xp_ref, xa_ref,
                      wlc_ref, wrc_ref, wlw_ref, wrw_ref, wlb_ref, wrb_ref,
                      blc_ref, brc_ref, blw_ref, brw_ref, blb_ref, brb_ref,
                      outp_ref, outa_ref):
    f32 = jnp.float32

    def mean(p0_ref, p1_ref, c0_ref, c1_ref):
        cnt = c0_ref[...] + c1_ref[...]
        return (p0_ref[...] + p1_ref[...]) / jnp.maximum(cnt, 1.0)

    mc = mean(sc0_ref, sc1_ref, cc0_ref, cc1_ref)
    mw = mean(sw0_ref, sw1_ref, cw0_ref, cw1_ref)
    mb = mean(sb0_ref, sb1_ref, cb0_ref, cb1_ref)
    outp_ref[...] = (
        jnp.dot(mc, wlc_ref[...], preferred_element_type=f32)
        + jnp.dot(mw, wlw_ref[...], preferred_element_type=f32)
        + jnp.dot(xp_ref[...], wrc_ref[...] + wrw_ref[...],
                  preferred_element_type=f32)
        + blc_ref[...] + brc_ref[...] + blw_ref[...] + brw_ref[...]
    )
    outa_ref[...] = (
        jnp.dot(mb, wlb_ref[...], preferred_element_type=f32)
        + jnp.dot(xa_ref[...], wrb_ref[...], preferred_element_type=f32)
        + blb_ref[...] + brb_ref[...]
    )


def _tc_epilogue(feats, cnts, xp, xa, ws, bs):
    f32 = jnp.float32
    bm = 1000
    grid = (N_NODE // bm,)
    feat = pl.BlockSpec((bm, D), lambda i: (i, 0))
    cnt = pl.BlockSpec((bm, 1), lambda i: (i, 0))
    wmat = pl.BlockSpec((D, D), lambda i: (0, 0))
    bvec = pl.BlockSpec((1, D), lambda i: (0, 0))
    sc0, sw0, sb0, sc1, sw1, sb1 = feats
    cc0, cw0, cb0, cc1, cw1, cb1 = cnts
    return pl.pallas_call(
        _tc_epilogue_body,
        grid=grid,
        in_specs=[feat, feat, feat, feat, feat, feat,
                  cnt, cnt, cnt, cnt, cnt, cnt, feat, feat,
                  wmat, wmat, wmat, wmat, wmat, wmat,
                  bvec, bvec, bvec, bvec, bvec, bvec],
        out_specs=[feat, feat],
        out_shape=[jax.ShapeDtypeStruct((N_NODE, D), f32),
                   jax.ShapeDtypeStruct((N_NODE, D), f32)],
    )(sc0, sc1, sw0, sw1, sb0, sb1, cc0, cc1, cw0, cw1, cb0, cb1,
      xp, xa, *ws, *bs)


def _unpack_counts(cnt_out):
    res = []
    for cpart in (cnt_out[:CROWS], cnt_out[CROWS:]):
        rows = cpart[:N_NODE // 8]
        lanes = rows.reshape(N_NODE // 8, 8, L)[:, :, 0]
        res.append(lanes.reshape(N_NODE, 1))
    return res


def kernel(x_paper, x_author, edge_index_cites, edge_index_writes,
           edge_index_written_by,
           Wl_cites, bl_cites, Wr_cites, br_cites,
           Wl_writes, bl_writes, Wr_writes, br_writes,
           Wl_wb, bl_wb, Wr_wb, br_wb):
    i32 = jnp.int32
    f32 = jnp.float32
    ec = edge_index_cites.astype(i32)
    ew = edge_index_writes.astype(i32)
    eb = edge_index_written_by.astype(i32)
    src_c, dst_c = _pad_edges(ec[0], ec[1], CHUNKS_CITES)
    src_w, dst_w = _pad_edges(ew[0], ew[1], CHUNKS_WRITES)
    src_b, dst_b = _pad_edges(eb[0], eb[1], CHUNKS_WRITES)

    zfeat = jnp.zeros((CHUNK, D), f32)
    onehot8 = (jnp.arange(D)[None, :] == (jnp.arange(8) * L)[:, None]
               ).astype(f32)

    sum_c, cnt_c, sum_w, cnt_w, sum_b, cnt_b = _sc_segment_sums(
        src_c, dst_c, src_w, dst_w, src_b, dst_b, x_paper, x_author,
        zfeat, onehot8)

    feats = (sum_c[:N_NODE], sum_w[:N_NODE], sum_b[:N_NODE],
             sum_c[PAD_N:PAD_N + N_NODE], sum_w[PAD_N:PAD_N + N_NODE],
             sum_b[PAD_N:PAD_N + N_NODE])
    cc0, cc1 = _unpack_counts(cnt_c)
    cw0, cw1 = _unpack_counts(cnt_w)
    cb0, cb1 = _unpack_counts(cnt_b)
    cnts = (cc0, cw0, cb0, cc1, cw1, cb1)
    ws = (Wl_cites, Wr_cites, Wl_writes, Wr_writes, Wl_wb, Wr_wb)
    bs = (bl_cites.reshape(1, D), br_cites.reshape(1, D),
          bl_writes.reshape(1, D), br_writes.reshape(1, D),
          bl_wb.reshape(1, D), br_wb.reshape(1, D))
    out_p, out_a = _tc_epilogue(feats, cnts, x_paper, x_author, ws, bs)
    return (out_p, out_a)


# speedup vs baseline: 3.6191x; 3.5852x over previous
import functools

import jax
import jax.numpy as jnp
from jax import lax
from jax.experimental import pallas as pl
from jax.experimental.pallas import tpu as pltpu
from jax.experimental.pallas import tpu_sc as plsc

N_NODE = 10000
D = 128
NC = 2
NS = 16
NW = NC * NS
L = 16
CHUNK = 64
PAD_N = 10112
ROWS_PER_SUB = PAD_N // NS
CROWS = 80

CHUNKS_CITES = 157
CHUNKS_WRITES = 79


def _pad_edges(src, dst, n_chunks):
    epad = NW * n_chunks * CHUNK
    pad = epad - src.shape[0]
    src = jnp.concatenate([src, jnp.zeros((pad,), jnp.int32)])
    dst = jnp.concatenate([dst, jnp.full((pad,), N_NODE, jnp.int32)])
    return src, dst


def _pieces(total, step):
    out, k = [], 0
    while k < total:
        out.append((k, min(step, total - k)))
        k += step
    return out


def _sc_segment_sums(src_c, dst_c, src_w, dst_w, src_b, dst_b, xp, xa,
                     zfeat, ident):
    mesh = plsc.VectorSubcoreMesh(core_axis_name="c", subcore_axis_name="s")
    f32 = jnp.float32

    @functools.partial(
        pl.kernel,
        mesh=mesh,
        out_type=[
            jax.ShapeDtypeStruct((NC * PAD_N, D), f32),
            jax.ShapeDtypeStruct((NC * CROWS, D), f32),
            jax.ShapeDtypeStruct((NC * PAD_N, D), f32),
            jax.ShapeDtypeStruct((NC * CROWS, D), f32),
            jax.ShapeDtypeStruct((NC * PAD_N, D), f32),
            jax.ShapeDtypeStruct((NC * CROWS, D), f32),
        ],
        scratch_types=[
            pltpu.VMEM((CHUNK,), jnp.int32),
            pltpu.VMEM((CHUNK,), jnp.int32),
            pltpu.VMEM((CHUNK,), jnp.int32),
            pltpu.VMEM((CHUNK,), jnp.int32),
            pltpu.VMEM((CHUNK,), jnp.int32),
            pltpu.VMEM((CHUNK,), jnp.int32),
            pltpu.VMEM((CHUNK,), jnp.int32),
            pltpu.VMEM((CHUNK,), jnp.int32),
            pltpu.VMEM((CHUNK, D), f32),
            pltpu.VMEM((CHUNK, D), f32),
            pltpu.VMEM((CHUNK, D), f32),
            pltpu.VMEM((CHUNK, D), f32),
            pltpu.VMEM_SHARED((PAD_N, D), f32),
            pltpu.VMEM_SHARED((CROWS, D), f32),
            pltpu.SemaphoreType.DMA,
            pltpu.SemaphoreType.DMA,
            pltpu.SemaphoreType.DMA,
            pltpu.SemaphoreType.DMA,
            pltpu.SemaphoreType.DMA,
            pltpu.SemaphoreType.DMA,
        ],
    )
    def sc_kernel(src_c_h, dst_c_h, src_w_h, dst_w_h, src_b_h, dst_b_h,
                  xp_h, xa_h, zfeat_h, ident_h,
                  o_sum_c, o_cnt_c, o_sum_w, o_cnt_w, o_sum_b, o_cnt_b,
                  sidx0, sidx1, didx0, didx1, drow0, drow1, dlane0, dlane1,
                  rows_f0, rows_f1, rows_c0, rows_c1, accum, cnt_acc,
                  sem_i0, sem_i1, sem_gf0, sem_gf1, sem_gc0, sem_gc1):
        c = lax.axis_index("c")
        s = lax.axis_index("s")
        wid = c * NS + s
        r0 = s * ROWS_PER_SUB
        fpieces = _pieces(ROWS_PER_SUB, CHUNK)
        slots = ((sidx0, didx0, drow0, dlane0, rows_f0, rows_c0,
                  sem_i0, sem_gf0, sem_gc0),
                 (sidx1, didx1, drow1, dlane1, rows_f1, rows_c1,
                  sem_i1, sem_gf1, sem_gc1))

        def idx_prefetch(src_h, dst_h, off, sl):
            sidx, didx = sl[0], sl[1]
            pltpu.async_copy(src_h.at[pl.ds(off, CHUNK)], sidx, sl[6])
            pltpu.async_copy(dst_h.at[pl.ds(off, CHUNK)], didx, sl[6])

        def idx_wait(src_h, dst_h, sl):
            pltpu.make_async_copy(src_h.at[pl.ds(0, CHUNK)], sl[0],
                                  sl[6]).wait()
            pltpu.make_async_copy(dst_h.at[pl.ds(0, CHUNK)], sl[1],
                                  sl[6]).wait()

        def compute_derived(sl):
            didx, drow, dlane = sl[1], sl[2], sl[3]
            for t in range(CHUNK // L):
                dv = didx[pl.ds(t * L, L)]
                drow[pl.ds(t * L, L)] = lax.shift_right_logical(dv, 7)
                dlane[pl.ds(t * L, L)] = lax.bitwise_and(dv, 127)

        def gather_issue(x_h, sl):
            pltpu.async_copy(x_h.at[sl[0]], sl[4], sl[7])
            pltpu.async_copy(ident_h.at[sl[3]], sl[5], sl[8])

        def gather_wait(sl):
            pltpu.make_async_copy(zfeat_h, sl[4], sl[7]).wait()
            pltpu.make_async_copy(zfeat_h, sl[5], sl[8]).wait()

        def run_relation(src_h, dst_h, x_h, out_sum, out_cnt, n):
            pltpu.sync_copy(zfeat_h, rows_f0)
            for (k, sz) in fpieces:
                pltpu.sync_copy(rows_f0.at[pl.ds(0, sz)],
                                accum.at[pl.ds(r0 + k, sz)])

            @pl.when(s < CROWS // 8)
            def _():
                pltpu.sync_copy(rows_f0.at[pl.ds(0, 8)],
                                cnt_acc.at[pl.ds(s * 8, 8)])

            plsc.subcore_barrier()
            base = wid * (n * CHUNK)

            pltpu.sync_copy(src_h.at[pl.ds(base, CHUNK)], sidx0)
            pltpu.sync_copy(dst_h.at[pl.ds(base, CHUNK)], didx0)
            compute_derived(slots[0])
            gather_issue(x_h, slots[0])
            idx_prefetch(src_h, dst_h,
                         pl.multiple_of(base + CHUNK, 8), slots[1])

            def iter_body(g, sl, sl_next):
                @pl.when(g < n)
                def _():
                    gather_wait(sl)

                    @pl.when(g + 1 < n)
                    def _():
                        idx_wait(src_h, dst_h, sl_next)
                        compute_derived(sl_next)
                        gather_issue(x_h, sl_next)

                    pltpu.sync_copy(sl[4], accum.at[sl[1]], add=True)
                    pltpu.sync_copy(sl[5], cnt_acc.at[sl[2]], add=True)

                    @pl.when(g + 2 < n)
                    def _():
                        off = pl.multiple_of(base + (g + 2) * CHUNK, 8)
                        idx_prefetch(src_h, dst_h, off, sl)

            def pair_body(gg, carry):
                iter_body(2 * gg, slots[0], slots[1])
                iter_body(2 * gg + 1, slots[1], slots[0])
                return carry

            lax.fori_loop(0, (n + 1) // 2, pair_body, 0)
            plsc.subcore_barrier()
            for (k, sz) in fpieces:
                pltpu.sync_copy(accum.at[pl.ds(r0 + k, sz)],
                                rows_f0.at[pl.ds(0, sz)])
                pltpu.sync_copy(rows_f0.at[pl.ds(0, sz)],
                                out_sum.at[pl.ds(c * PAD_N + r0 + k, sz)])

            @pl.when(s < CROWS // 8)
            def _():
                pltpu.sync_copy(cnt_acc.at[pl.ds(s * 8, 8)],
                                rows_c0.at[pl.ds(0, 8)])
                pltpu.sync_copy(rows_c0.at[pl.ds(0, 8)],
                                out_cnt.at[pl.ds(c * CROWS + s * 8, 8)])

            plsc.subcore_barrier()

        run_relation(src_c_h, dst_c_h, xp_h, o_sum_c, o_cnt_c, CHUNKS_CITES)
        run_relation(src_w_h, dst_w_h, xa_h, o_sum_w, o_cnt_w, CHUNKS_WRITES)
        run_relation(src_b_h, dst_b_h, xp_h, o_sum_b, o_cnt_b, CHUNKS_WRITES)

    return sc_kernel(src_c, dst_c, src_w, dst_w, src_b, dst_b, xp, xa,
                     zfeat, ident)


def _tc_epilogue_body(sc0_ref, sc1_ref, sw0_ref, sw1_ref, sb0_ref, sb1_ref,
                      cc0_ref, cc1_ref, cw0_ref, cw1_ref, cb0_ref, cb1_ref,
                      xp_ref, xa_ref,
                      wlc_ref, wrc_ref, wlw_ref, wrw_ref, wlb_ref, wrb_ref,
                      blc_ref, brc_ref, blw_ref, brw_ref, blb_ref, brb_ref,
                      outp_ref, outa_ref):
    f32 = jnp.float32

    def mean(p0_ref, p1_ref, c0_ref, c1_ref):
        cnt = c0_ref[...] + c1_ref[...]
        return (p0_ref[...] + p1_ref[...]) / jnp.maximum(cnt, 1.0)

    mc = mean(sc0_ref, sc1_ref, cc0_ref, cc1_ref)
    mw = mean(sw0_ref, sw1_ref, cw0_ref, cw1_ref)
    mb = mean(sb0_ref, sb1_ref, cb0_ref, cb1_ref)
    outp_ref[...] = (
        jnp.dot(mc, wlc_ref[...], preferred_element_type=f32)
        + jnp.dot(mw, wlw_ref[...], preferred_element_type=f32)
        + jnp.dot(xp_ref[...], wrc_ref[...] + wrw_ref[...],
                  preferred_element_type=f32)
        + blc_ref[...] + brc_ref[...] + blw_ref[...] + brw_ref[...]
    )
    outa_ref[...] = (
        jnp.dot(mb, wlb_ref[...], preferred_element_type=f32)
        + jnp.dot(xa_ref[...], wrb_ref[...], preferred_element_type=f32)
        + blb_ref[...] + brb_ref[...]
    )


def _tc_epilogue(feats, cnts, xp, xa, ws, bs):
    f32 = jnp.float32
    bm = 1000
    grid = (N_NODE // bm,)
    feat = pl.BlockSpec((bm, D), lambda i: (i, 0))
    cnt = pl.BlockSpec((bm, 1), lambda i: (i, 0))
    wmat = pl.BlockSpec((D, D), lambda i: (0, 0))
    bvec = pl.BlockSpec((1, D), lambda i: (0, 0))
    sc0, sw0, sb0, sc1, sw1, sb1 = feats
    cc0, cw0, cb0, cc1, cw1, cb1 = cnts
    return pl.pallas_call(
        _tc_epilogue_body,
        grid=grid,
        in_specs=[feat, feat, feat, feat, feat, feat,
                  cnt, cnt, cnt, cnt, cnt, cnt, feat, feat,
                  wmat, wmat, wmat, wmat, wmat, wmat,
                  bvec, bvec, bvec, bvec, bvec, bvec],
        out_specs=[feat, feat],
        out_shape=[jax.ShapeDtypeStruct((N_NODE, D), f32),
                   jax.ShapeDtypeStruct((N_NODE, D), f32)],
    )(sc0, sc1, sw0, sw1, sb0, sb1, cc0, cc1, cw0, cw1, cb0, cb1,
      xp, xa, *ws, *bs)


def _unpack_counts(cnt_out):
    res = []
    for cpart in (cnt_out[:CROWS], cnt_out[CROWS:]):
        res.append(cpart.reshape(CROWS * D)[:N_NODE].reshape(N_NODE, 1))
    return res


def kernel(x_paper, x_author, edge_index_cites, edge_index_writes,
           edge_index_written_by,
           Wl_cites, bl_cites, Wr_cites, br_cites,
           Wl_writes, bl_writes, Wr_writes, br_writes,
           Wl_wb, bl_wb, Wr_wb, br_wb):
    i32 = jnp.int32
    f32 = jnp.float32
    ec = edge_index_cites.astype(i32)
    ew = edge_index_writes.astype(i32)
    eb = edge_index_written_by.astype(i32)
    src_c, dst_c = _pad_edges(ec[0], ec[1], CHUNKS_CITES)
    src_w, dst_w = _pad_edges(ew[0], ew[1], CHUNKS_WRITES)
    src_b, dst_b = _pad_edges(eb[0], eb[1], CHUNKS_WRITES)

    zfeat = jnp.zeros((CHUNK, D), f32)
    ident = jnp.eye(D, dtype=f32)

    sum_c, cnt_c, sum_w, cnt_w, sum_b, cnt_b = _sc_segment_sums(
        src_c, dst_c, src_w, dst_w, src_b, dst_b, x_paper, x_author,
        zfeat, ident)

    feats = (sum_c[:N_NODE], sum_w[:N_NODE], sum_b[:N_NODE],
             sum_c[PAD_N:PAD_N + N_NODE], sum_w[PAD_N:PAD_N + N_NODE],
             sum_b[PAD_N:PAD_N + N_NODE])
    cc0, cc1 = _unpack_counts(cnt_c)
    cw0, cw1 = _unpack_counts(cnt_w)
    cb0, cb1 = _unpack_counts(cnt_b)
    cnts = (cc0, cw0, cb0, cc1, cw1, cb1)
    ws = (Wl_cites, Wr_cites, Wl_writes, Wr_writes, Wl_wb, Wr_wb)
    bs = (bl_cites.reshape(1, D), br_cites.reshape(1, D),
          bl_writes.reshape(1, D), br_writes.reshape(1, D),
          bl_wb.reshape(1, D), br_wb.reshape(1, D))
    out_p, out_a = _tc_epilogue(feats, cnts, x_paper, x_author, ws, bs)
    return (out_p, out_a)
